# baseline (speedup 1.0000x reference)
import functools

import jax
import jax.numpy as jnp
from jax import lax
from jax.experimental import pallas as pl
from jax.experimental.pallas import tpu as pltpu
from jax.experimental.pallas import tpu_sc as plsc

B, S, HID = 1, 2048, 768
ENT, D = 9, 64
HALF = D // 2
MT = 2
BM = S // MT


@functools.lru_cache(maxsize=None)
def _build_sc_gather():
    info = plsc.get_sparse_core_info()
    nc, ns = info.num_cores, info.num_subcores
    nw = nc * ns
    rows_per_w = S // nw
    mesh = plsc.VectorSubcoreMesh(core_axis_name="c", subcore_axis_name="s")

    @functools.partial(
        pl.kernel,
        out_type=jax.ShapeDtypeStruct((S, HID), jnp.float32),
        mesh=mesh,
        scratch_types=[
            pltpu.VMEM((rows_per_w,), jnp.int32),
            pltpu.VMEM((rows_per_w, HID), jnp.float32),
            pltpu.SemaphoreType.DMA,
        ],
    )
    def gather_kernel(ids_hbm, table_hbm, out_hbm, idx_v, rows_v, sem):
        wid = lax.axis_index("s") * nc + lax.axis_index("c")
        base = wid * rows_per_w
        pltpu.sync_copy(ids_hbm.at[pl.ds(base, rows_per_w)], idx_v)
        pltpu.async_copy(table_hbm.at[idx_v], rows_v, sem).wait()
        pltpu.sync_copy(rows_v, out_hbm.at[pl.ds(base, rows_per_w)])

    return gather_kernel


def _rotate_half(x):
    return jnp.concatenate([-x[:, HALF:], x[:, :HALF]], axis=1)


def _proj_body(hid_ref, w_ref, b_ref, dtw_ref, ttf_ref, cos_ref, sin_ref,
               qr_ref, kr_ref):
    ph = jnp.dot(hid_ref[...].astype(jnp.bfloat16), w_ref[0],
                 preferred_element_type=jnp.float32)
    ph = ph + b_ref[0] + ttf_ref[...] * dtw_ref[0]
    cos = cos_ref[...]
    sin = sin_ref[...]
    q = ph[:, :D]
    k = ph[:, D:]
    qr_ref[0] = (q * cos + _rotate_half(q) * sin).astype(jnp.bfloat16)
    kr_ref[0] = (k * cos + _rotate_half(k) * sin).astype(jnp.bfloat16)


def _proj_call(hidden, w_all, b_all, dtw_all, ttf, cos_h, sin_h):
    return pl.pallas_call(
        _proj_body,
        grid=(ENT,),
        in_specs=[
            pl.BlockSpec((S, HID), lambda h: (0, 0)),
            pl.BlockSpec((1, HID, 2 * D), lambda h: (h, 0, 0)),
            pl.BlockSpec((1, 1, 2 * D), lambda h: (h, 0, 0)),
            pl.BlockSpec((1, 1, 2 * D), lambda h: (h, 0, 0)),
            pl.BlockSpec((S, 1), lambda h: (0, 0)),
            pl.BlockSpec((S, D), lambda h: (0, 0)),
            pl.BlockSpec((S, D), lambda h: (0, 0)),
        ],
        out_specs=[
            pl.BlockSpec((1, S, D), lambda h: (h, 0, 0)),
            pl.BlockSpec((1, S, D), lambda h: (h, 0, 0)),
        ],
        out_shape=[
            jax.ShapeDtypeStruct((ENT, S, D), jnp.bfloat16),
            jax.ShapeDtypeStruct((ENT, S, D), jnp.bfloat16),
        ],
        compiler_params=pltpu.CompilerParams(
            dimension_semantics=("parallel",)),
    )(hidden, w_all, b_all, dtw_all, ttf, cos_h, sin_h)


def _dot_body(qr_ref, kr_ref, out_ref):
    out_ref[0] = lax.dot_general(qr_ref[0], kr_ref[0],
                                 (((1,), (1,)), ((), ())),
                                 preferred_element_type=jnp.float32)


def _dot_call(qr, kr):
    return pl.pallas_call(
        _dot_body,
        grid=(ENT, MT),
        in_specs=[
            pl.BlockSpec((1, BM, D), lambda h, m: (h, m, 0)),
            pl.BlockSpec((1, S, D), lambda h, m: (h, 0, 0)),
        ],
        out_specs=pl.BlockSpec((1, BM, S), lambda h, m: (h, m, 0)),
        out_shape=jax.ShapeDtypeStruct((ENT, S, S), jnp.float32),
        compiler_params=pltpu.CompilerParams(
            dimension_semantics=("parallel", "parallel")),
    )(qr, kr)


def _prep(token_type_ids, type_table, dense_W, dense_b):
    perm = jnp.concatenate([jnp.arange(0, D, 2), jnp.arange(1, D, 2)])
    kscale = 1.0 / (D ** 0.5)

    w3 = dense_W.reshape(HID, ENT, 2 * D)
    wq = w3[..., :D][..., perm]
    wk = w3[..., D:][..., perm] * kscale
    w_all = jnp.concatenate([wq, wk], axis=-1).transpose(1, 0, 2)
    w_all = w_all.astype(jnp.bfloat16)

    b_eff = dense_b + type_table[0] @ dense_W
    dtw = (type_table[1] - type_table[0]) @ dense_W

    def head_perm(v):
        v3 = v.reshape(ENT, 2 * D)
        vq = v3[:, :D][:, perm]
        vk = v3[:, D:][:, perm] * kscale
        return jnp.concatenate([vq, vk], axis=-1)[:, None, :]

    b_all = head_perm(b_eff)
    dtw_all = head_perm(dtw)

    pos = jnp.arange(S, dtype=jnp.float32)[:, None]
    freq = jnp.power(10000.0, -2.0 * jnp.arange(HALF, dtype=jnp.float32) / D)
    ang = pos * freq
    cos_h = jnp.tile(jnp.cos(ang), (1, 2))
    sin_h = jnp.tile(jnp.sin(ang), (1, 2))

    ttf = token_type_ids.reshape(S, 1).astype(jnp.float32)
    return w_all, b_all, dtw_all, ttf, cos_h, sin_h


def kernel(input_ids, attention_mask, token_type_ids, emb_table, type_table,
           dense_W, dense_b):
    ids = input_ids.reshape(S)
    hidden = _build_sc_gather()(ids, emb_table)
    w_all, b_all, dtw_all, ttf, cos_h, sin_h = _prep(
        token_type_ids, type_table, dense_W, dense_b)
    qr, kr = _proj_call(hidden, w_all, b_all, dtw_all, ttf, cos_h, sin_h)
    logits = _dot_call(qr, kr)
    return logits.reshape(B, ENT, S, S)
